# baseline (speedup 1.0000x reference)
import functools

import jax
import jax.numpy as jnp
from jax import lax
from jax.experimental import pallas as pl
from jax.experimental.pallas import tpu as pltpu
from jax.experimental.pallas import tpu_sc as plsc


def kernel(x, pe):
    batch, seq_len, d_model = x.shape
    info = plsc.get_sparse_core_info()
    nw = info.num_cores * info.num_subcores
    seq_per_w = seq_len // nw
    rows = 16
    chunk = rows * d_model
    n_chunks = seq_per_w // rows
    lanes = info.num_lanes

    xf = x.reshape(batch * seq_len * d_model)
    pef = pe.reshape(pe.shape[0] * d_model)

    mesh = plsc.VectorSubcoreMesh(core_axis_name="c", subcore_axis_name="s")

    @functools.partial(
        pl.kernel,
        out_type=jax.ShapeDtypeStruct((batch * seq_len * d_model,), jnp.float32),
        mesh=mesh,
        scratch_types=[
            pltpu.VMEM((chunk,), jnp.float32),
            pltpu.VMEM((chunk,), jnp.float32),
        ],
    )
    def sc_add(x_hbm, pe_hbm, out_hbm, x_buf, pe_buf):
        wid = lax.axis_index("s") * info.num_cores + lax.axis_index("c")
        s0 = wid * seq_per_w

        def chunk_body(c, _):
            pe_off = (s0 + c * rows) * d_model
            pltpu.sync_copy(pe_hbm.at[pl.ds(pe_off, chunk)], pe_buf)

            def batch_body(b, _):
                x_off = b * (seq_len * d_model) + pe_off
                pltpu.sync_copy(x_hbm.at[pl.ds(x_off, chunk)], x_buf)

                def vec_body(i, _):
                    sl = pl.ds(i * lanes, lanes)
                    x_buf[sl] = x_buf[sl] + pe_buf[sl]
                    return 0

                lax.fori_loop(0, chunk // lanes, vec_body, 0)
                pltpu.sync_copy(x_buf, out_hbm.at[pl.ds(x_off, chunk)])
                return 0

            lax.fori_loop(0, batch, batch_body, 0)
            return 0

        lax.fori_loop(0, n_chunks, chunk_body, 0)

    return sc_add(xf, pef).reshape(x.shape)


# speedup vs baseline: 1.0664x; 1.0664x over previous
import functools

import jax
import jax.numpy as jnp
from jax import lax
from jax.experimental import pallas as pl
from jax.experimental.pallas import tpu as pltpu
from jax.experimental.pallas import tpu_sc as plsc


def kernel(x, pe):
    batch, seq_len, d_model = x.shape
    info = plsc.get_sparse_core_info()
    nw = info.num_cores * info.num_subcores
    seq_per_w = seq_len // nw
    rows = 16
    chunk = rows * d_model
    n_chunks = seq_per_w // rows
    lanes = info.num_lanes
    unroll = 8

    xf = x.reshape(batch * seq_len * d_model)
    pef = pe.reshape(pe.shape[0] * d_model)

    mesh = plsc.VectorSubcoreMesh(core_axis_name="c", subcore_axis_name="s")

    @functools.partial(
        pl.kernel,
        out_type=jax.ShapeDtypeStruct((batch * seq_len * d_model,), jnp.float32),
        mesh=mesh,
        scratch_types=(
            [pltpu.VMEM((batch, chunk), jnp.float32), pltpu.VMEM((chunk,), jnp.float32)]
            + [pltpu.SemaphoreType.DMA] * (2 * batch)
        ),
    )
    def sc_add(x_hbm, pe_hbm, out_hbm, xb, peb, *sems):
        sin, sout = sems[:batch], sems[batch:]
        wid = lax.axis_index("s") * info.num_cores + lax.axis_index("c")
        s0 = wid * seq_per_w

        def group(c, _):
            pe_off = (s0 + c * rows) * d_model
            in_handles = []
            for b in range(batch):
                x_off = b * seq_len * d_model + pe_off
                in_handles.append(
                    pltpu.async_copy(x_hbm.at[pl.ds(x_off, chunk)], xb.at[b], sin[b])
                )
            pltpu.sync_copy(pe_hbm.at[pl.ds(pe_off, chunk)], peb)
            out_handles = []
            for b in range(batch):
                x_off = b * seq_len * d_model + pe_off
                in_handles[b].wait()

                def vec_body(i, _, b=b):
                    base = i * (lanes * unroll)
                    for u in range(unroll):
                        sl = pl.ds(base + u * lanes, lanes)
                        xb[b, sl] = xb[b, sl] + peb[sl]
                    return 0

                lax.fori_loop(0, chunk // (lanes * unroll), vec_body, 0)
                out_handles.append(
                    pltpu.async_copy(xb.at[b], out_hbm.at[pl.ds(x_off, chunk)], sout[b])
                )
            for h in out_handles:
                h.wait()
            return 0

        lax.fori_loop(0, n_chunks, group, 0)

    return sc_add(xf, pef).reshape(x.shape)


# speedup vs baseline: 7.9713x; 7.4748x over previous
import jax
import jax.numpy as jnp
from jax.experimental import pallas as pl


def _pe_add_kernel(x_ref, pe_ref, o_ref):
    o_ref[...] = x_ref[...] + pe_ref[...][None, :, :]


def kernel(x, pe):
    batch, seq_len, d_model = x.shape
    s_blk = 1024
    b_blk = 2
    grid = (seq_len // s_blk, batch // b_blk)
    return pl.pallas_call(
        _pe_add_kernel,
        grid=grid,
        in_specs=[
            pl.BlockSpec((b_blk, s_blk, d_model), lambda s, b: (b, s, 0)),
            pl.BlockSpec((s_blk, d_model), lambda s, b: (s, 0)),
        ],
        out_specs=pl.BlockSpec((b_blk, s_blk, d_model), lambda s, b: (b, s, 0)),
        out_shape=jax.ShapeDtypeStruct(x.shape, x.dtype),
    )(x, pe)
